# baseline (speedup 1.0000x reference)
import functools

import jax
import jax.numpy as jnp
from jax import lax
from jax.experimental import pallas as pl
from jax.experimental.pallas import tpu as pltpu
from jax.experimental.pallas import tpu_sc as plsc

PAD = 1
SEQ = 4096
BATCH = 4
DIM = 1024
TOTAL = BATCH * SEQ
NUM_TILES = 32
TOK_PER_TILE = TOTAL // NUM_TILES
CHUNKS_PER_ROW = SEQ // TOK_PER_TILE
CH = 16
NCH = TOK_PER_TILE // CH
L = 16
NBUF = 5


def _sc_body(inp_hbm, table_hbm, out_hbm, tokens_v, idx_v, *rest):
  bufs = rest[:NBUF]
  gsems = rest[NBUF:2 * NBUF]
  osems = rest[2 * NBUF:3 * NBUF]
  nc = 2
  wid = lax.axis_index("s") * nc + lax.axis_index("c")
  row = wid // CHUNKS_PER_ROW
  chunk = wid % CHUNKS_PER_ROW
  rbase = row * SEQ

  pltpu.sync_copy(inp_hbm.at[pl.ds(rbase, SEQ)], tokens_v)

  nvecs = chunk * (TOK_PER_TILE // L)

  def obody(i, acc):
    for u in range(4):
      v = tokens_v[pl.ds(i * (4 * L) + u * L, L)]
      acc = acc + jnp.where(v != PAD, jnp.int32(1), jnp.int32(0))
    return acc

  accv = lax.fori_loop(0, nvecs // 4, obody, jnp.zeros((L,), jnp.int32))
  offset = jnp.sum(accv)

  base = chunk * TOK_PER_TILE
  out_base = rbase + base
  carry = [offset]

  def make_idx(g):
    v = tokens_v[pl.ds(base + g * L, L)]
    m = jnp.where(v != PAD, jnp.int32(1), jnp.int32(0))
    cs = jnp.cumsum(m) + carry[0]
    pos = jnp.where(v != PAD, cs, jnp.int32(0)) + PAD
    idx_v[pl.ds(g * L, L)] = pos
    carry[0] = cs[L - 1]

  gh = [None] * NBUF
  oh = [None] * NBUF

  def fire_gather(g):
    b = g % NBUF
    make_idx(g)
    gh[b] = pltpu.async_copy(
        table_hbm.at[idx_v.at[pl.ds(g * CH, CH)]], bufs[b], gsems[b])

  for g in range(min(NBUF - 1, NCH)):
    fire_gather(g)
  for g in range(NCH):
    b = g % NBUF
    gh[b].wait()
    oh[b] = pltpu.async_copy(
        bufs[b], out_hbm.at[pl.ds(out_base + g * CH, CH)], osems[b])
    nxt = g + NBUF - 1
    if nxt < NCH:
      nb = nxt % NBUF
      if oh[nb] is not None:
        oh[nb].wait()
      fire_gather(nxt)
  for g in range(max(0, NCH - (NBUF - 1)), NCH):
    oh[g % NBUF].wait()


@jax.jit
def _lookup(inp_flat, table):
  mesh = plsc.VectorSubcoreMesh(core_axis_name="c", subcore_axis_name="s")
  k = functools.partial(
      pl.kernel,
      mesh=mesh,
      compiler_params=pltpu.CompilerParams(needs_layout_passes=False),
      out_type=jax.ShapeDtypeStruct((TOTAL, DIM), jnp.float32),
      scratch_types=(
          [pltpu.VMEM((SEQ,), jnp.int32), pltpu.VMEM((TOK_PER_TILE,), jnp.int32)]
          + [pltpu.VMEM((CH, DIM), jnp.float32)] * NBUF
          + [pltpu.SemaphoreType.DMA] * (2 * NBUF)
      ),
  )(_sc_body)
  return k(inp_flat, table)


def kernel(input, table):
  out = _lookup(input.reshape(-1), table)
  return out.reshape(BATCH, SEQ, DIM)
